# baseline (speedup 1.0000x reference)
import functools

import jax
import jax.numpy as jnp
from jax import lax
from jax.experimental import pallas as pl
from jax.experimental.pallas import tpu as pltpu
from jax.experimental.pallas import tpu_sc as plsc

D = 2048
T = 512
NC, NS = 2, 16
NW = NC * NS
S = 2


def _make_row_permute_gather(n_tok, n_slots, d, ch):
    per_w = n_tok // NW
    nchunks = per_w // ch
    assert n_tok % NW == 0 and per_w % ch == 0 and ch % 8 == 0

    mesh = plsc.VectorSubcoreMesh(core_axis_name="c", subcore_axis_name="s")

    nbuf = 3

    @functools.partial(
        pl.kernel,
        out_type=jax.ShapeDtypeStruct((n_slots, d), jnp.float32),
        mesh=mesh,
        scratch_types=[
            pltpu.VMEM((nchunks, ch), jnp.int32),
            pltpu.VMEM((nchunks, ch), jnp.int32),
        ] + [pltpu.VMEM((ch, d), jnp.float32) for _ in range(nbuf)] + [
            pltpu.SemaphoreType.DMA,
            pltpu.SemaphoreType.DMA,
        ],
    )
    def gather(src_hbm, idx_hbm, pos_hbm, out_hbm, idx_v, pos_v, *rest):
        bufs, gsem, ssem = rest[:nbuf], rest[nbuf], rest[nbuf + 1]
        wid = lax.axis_index("s") * NC + lax.axis_index("c")
        pltpu.sync_copy(idx_hbm.at[wid], idx_v)
        pltpu.sync_copy(pos_hbm.at[wid], pos_v)
        for j in range(min(nbuf - 1, nchunks)):
            pltpu.async_copy(src_hbm.at[idx_v.at[j]], bufs[j % nbuf], gsem)
        for j in range(nchunks):
            pltpu.make_async_copy(src_hbm.at[idx_v.at[0]], bufs[j % nbuf], gsem).wait()
            g = j + nbuf - 1
            if g < nchunks:
                if j >= 1:
                    pltpu.make_async_copy(
                        bufs[(j - 1) % nbuf], out_hbm.at[pos_v.at[0]], ssem).wait()
                pltpu.async_copy(src_hbm.at[idx_v.at[g]], bufs[g % nbuf], gsem)
            pltpu.async_copy(bufs[j % nbuf], out_hbm.at[pos_v.at[j]], ssem)
        for j in range(min(nbuf, nchunks)):
            pltpu.make_async_copy(bufs[0], out_hbm.at[pos_v.at[0]], ssem).wait()

    return gather


def _make_row_gather_into(n_seg_rows, d, ch, row_base):
    per_w = n_seg_rows // NW
    nchunks = per_w // ch
    assert n_seg_rows % NW == 0 and per_w % ch == 0 and ch % 8 == 0

    mesh = plsc.VectorSubcoreMesh(core_axis_name="c", subcore_axis_name="s")

    nbuf = 3

    @functools.partial(
        pl.kernel,
        out_type=(),
        mesh=mesh,
        scratch_types=[
            pltpu.VMEM((nchunks, ch), jnp.int32),
        ] + [pltpu.VMEM((ch, d), jnp.float32) for _ in range(nbuf)] + [
            pltpu.SemaphoreType.DMA,
        ],
    )
    def gather(src_hbm, idx_hbm, dst_hbm, idx_v, *rest):
        bufs, gsem = rest[:nbuf], rest[nbuf]
        wid = lax.axis_index("s") * NC + lax.axis_index("c")
        base = row_base + wid * per_w
        pltpu.sync_copy(idx_hbm.at[wid], idx_v)
        for j in range(min(nbuf - 1, nchunks)):
            pltpu.async_copy(src_hbm.at[idx_v.at[j]], bufs[j % nbuf], gsem)
        for j in range(nchunks):
            pltpu.make_async_copy(src_hbm.at[idx_v.at[0]], bufs[j % nbuf], gsem).wait()
            g = j + nbuf - 1
            if g < nchunks:
                pltpu.async_copy(src_hbm.at[idx_v.at[g]], bufs[g % nbuf], gsem)
            pltpu.sync_copy(bufs[j % nbuf], dst_hbm.at[pl.ds(base + j * ch, ch)])

    return gather


def _bf16_cast_pair(w0, w1):
    def body(a_ref, b_ref, oa_ref, ob_ref):
        oa_ref[...] = a_ref[...].astype(jnp.bfloat16)
        ob_ref[...] = b_ref[...].astype(jnp.bfloat16)

    blk = D // 4
    return pl.pallas_call(
        body,
        grid=(4,),
        in_specs=[pl.BlockSpec((blk, D), lambda i: (i, 0)),
                  pl.BlockSpec((blk, D), lambda i: (i, 0))],
        out_specs=[pl.BlockSpec((blk, D), lambda i: (i, 0)),
                   pl.BlockSpec((blk, D), lambda i: (i, 0))],
        out_shape=[jax.ShapeDtypeStruct((D, D), jnp.bfloat16)] * 2,
    )(w0, w1)


def _routed_matmul_split(flags, x_sc, x_tc, w0, w1):
    kt = x_sc.shape[0] // T
    ntiles = kt + x_tc.shape[0] // T

    def body(flags_ref, xsc_ref, xtc_ref, w0_ref, w1_ref, o_ref):
        t = pl.program_id(0)
        f = flags_ref[t]
        xb = jnp.where(t < kt, xsc_ref[...], xtc_ref[...]).astype(jnp.bfloat16)

        @pl.when(f == 0)
        def _():
            o_ref[...] = lax.dot_general(
                xb, w0_ref[...], (((1,), (1,)), ((), ())),
                preferred_element_type=jnp.float32)

        @pl.when(f != 0)
        def _():
            o_ref[...] = lax.dot_general(
                xb, w1_ref[...], (((1,), (1,)), ((), ())),
                preferred_element_type=jnp.float32)

    grid_spec = pltpu.PrefetchScalarGridSpec(
        num_scalar_prefetch=1,
        grid=(ntiles,),
        in_specs=[
            pl.BlockSpec((T, D), lambda t, flags: (jnp.minimum(t, kt - 1), 0)),
            pl.BlockSpec((T, D), lambda t, flags: (jnp.maximum(t - kt, 0), 0)),
            pl.BlockSpec((D, D), lambda t, flags: (0, 0)),
            pl.BlockSpec((D, D), lambda t, flags: (0, 0)),
        ],
        out_specs=pl.BlockSpec((T, D), lambda t, flags: (t, 0)),
    )
    return pl.pallas_call(
        body,
        grid_spec=grid_spec,
        out_shape=jax.ShapeDtypeStruct((ntiles * T, D), jnp.float32),
    )(flags, x_sc, x_tc, w0, w1)


def _routed_matmul(flags, x, w0, w1):
    n = x.shape[0]
    ntiles = n // T

    def body(flags_ref, x_ref, w0_ref, w1_ref, o_ref):
        f = flags_ref[pl.program_id(0)]
        xb = x_ref[...].astype(jnp.bfloat16)

        @pl.when(f == 0)
        def _():
            o_ref[...] = lax.dot_general(
                xb, w0_ref[...], (((1,), (1,)), ((), ())),
                preferred_element_type=jnp.float32)

        @pl.when(f != 0)
        def _():
            o_ref[...] = lax.dot_general(
                xb, w1_ref[...], (((1,), (1,)), ((), ())),
                preferred_element_type=jnp.float32)

    grid_spec = pltpu.PrefetchScalarGridSpec(
        num_scalar_prefetch=1,
        grid=(ntiles,),
        in_specs=[
            pl.BlockSpec((T, D), lambda t, flags: (t, 0)),
            pl.BlockSpec((D, D), lambda t, flags: (0, 0)),
            pl.BlockSpec((D, D), lambda t, flags: (0, 0)),
        ],
        out_specs=pl.BlockSpec((T, D), lambda t, flags: (t, 0)),
    )
    return pl.pallas_call(
        body,
        grid_spec=grid_spec,
        out_shape=jax.ShapeDtypeStruct((n, D), jnp.float32),
    )(flags, x, w0, w1)


SIZES = (6144, 6144, 4096)


def kernel(input_ids, role_mask, table, W0, W1):
    b, l = input_ids.shape
    n = b * l
    assert sum(SIZES) == n

    ids_f = input_ids.reshape(-1).astype(jnp.int32)
    is0_f = role_mask.reshape(-1) == 0

    w0b, w1b = _bf16_cast_pair(W0, W1)

    out_ref = jax.new_ref(lax.empty((n, D), jnp.float32))

    off = 0
    for size in SIZES:
        npad = size + T
        ntiles = npad // T
        is0 = is0_f[off:off + size]
        i0 = is0.astype(jnp.int32)
        r0 = jnp.cumsum(i0) - 1
        r1 = jnp.cumsum(1 - i0) - 1
        c0 = jnp.sum(i0)
        start1 = (c0 // T) * T + T
        pos = jnp.where(is0, r0, start1 + r1).astype(jnp.int32)
        flags = (jnp.arange(ntiles, dtype=jnp.int32) >= (c0 // T + 1)).astype(jnp.int32)

        x_s = _make_row_permute_gather(size, npad, D, 16)(
            table, ids_f[off:off + size].reshape(NW, -1, 16),
            pos.reshape(NW, -1, 16))
        y_s = _routed_matmul(flags, x_s, w0b, w1b)
        _make_row_gather_into(size, D, 16, off)(y_s, pos.reshape(NW, -1, 16), out_ref)
        off += size

    return out_ref[...].reshape(b, l, D)


# speedup vs baseline: 1.0455x; 1.0455x over previous
import functools

import jax
import jax.numpy as jnp
from jax import lax
from jax.experimental import pallas as pl
from jax.experimental.pallas import tpu as pltpu
from jax.experimental.pallas import tpu_sc as plsc

D = 2048
T = 512
NC, NS = 2, 16
NW = NC * NS
S = 2


def _make_row_permute_gather(n_tok, n_slots, d, ch):
    per_w = n_tok // NW
    nchunks = per_w // ch
    assert n_tok % NW == 0 and per_w % ch == 0 and ch % 8 == 0

    mesh = plsc.VectorSubcoreMesh(core_axis_name="c", subcore_axis_name="s")

    nbuf = 3

    @functools.partial(
        pl.kernel,
        out_type=jax.ShapeDtypeStruct((n_slots, d), jnp.float32),
        mesh=mesh,
        scratch_types=[
            pltpu.VMEM((nchunks, ch), jnp.int32),
            pltpu.VMEM((nchunks, ch), jnp.int32),
        ] + [pltpu.VMEM((ch, d), jnp.float32) for _ in range(nbuf)] + [
            pltpu.SemaphoreType.DMA,
            pltpu.SemaphoreType.DMA,
        ],
    )
    def gather(src_hbm, idx_hbm, pos_hbm, out_hbm, idx_v, pos_v, *rest):
        bufs, gsem, ssem = rest[:nbuf], rest[nbuf], rest[nbuf + 1]
        wid = lax.axis_index("s") * NC + lax.axis_index("c")
        pltpu.sync_copy(idx_hbm.at[wid], idx_v)
        pltpu.sync_copy(pos_hbm.at[wid], pos_v)
        for j in range(min(nbuf - 1, nchunks)):
            pltpu.async_copy(src_hbm.at[idx_v.at[j]], bufs[j % nbuf], gsem)
        for j in range(nchunks):
            pltpu.make_async_copy(src_hbm.at[idx_v.at[0]], bufs[j % nbuf], gsem).wait()
            g = j + nbuf - 1
            if g < nchunks:
                if j >= 1:
                    pltpu.make_async_copy(
                        bufs[(j - 1) % nbuf], out_hbm.at[pos_v.at[0]], ssem).wait()
                pltpu.async_copy(src_hbm.at[idx_v.at[g]], bufs[g % nbuf], gsem)
            pltpu.async_copy(bufs[j % nbuf], out_hbm.at[pos_v.at[j]], ssem)
        for j in range(min(nbuf, nchunks)):
            pltpu.make_async_copy(bufs[0], out_hbm.at[pos_v.at[0]], ssem).wait()

    return gather


def _make_row_gather_into(n_seg_rows, d, ch, row_base):
    per_w = n_seg_rows // NW
    nchunks = per_w // ch
    assert n_seg_rows % NW == 0 and per_w % ch == 0 and ch % 8 == 0

    mesh = plsc.VectorSubcoreMesh(core_axis_name="c", subcore_axis_name="s")

    nbuf = 3

    @functools.partial(
        pl.kernel,
        out_type=(),
        mesh=mesh,
        scratch_types=[
            pltpu.VMEM((nchunks, ch), jnp.int32),
        ] + [pltpu.VMEM((ch, d), jnp.float32) for _ in range(nbuf)] + [
            pltpu.SemaphoreType.DMA,
        ],
    )
    def gather(src_hbm, idx_hbm, dst_hbm, idx_v, *rest):
        bufs, gsem = rest[:nbuf], rest[nbuf]
        wid = lax.axis_index("s") * NC + lax.axis_index("c")
        base = row_base + wid * per_w
        pltpu.sync_copy(idx_hbm.at[wid], idx_v)
        for j in range(min(nbuf - 1, nchunks)):
            pltpu.async_copy(src_hbm.at[idx_v.at[j]], bufs[j % nbuf], gsem)
        for j in range(nchunks):
            pltpu.make_async_copy(src_hbm.at[idx_v.at[0]], bufs[j % nbuf], gsem).wait()
            g = j + nbuf - 1
            if g < nchunks:
                pltpu.async_copy(src_hbm.at[idx_v.at[g]], bufs[g % nbuf], gsem)
            pltpu.sync_copy(bufs[j % nbuf], dst_hbm.at[pl.ds(base + j * ch, ch)])

    return gather


def _bf16_cast_pair(w0, w1):
    def body(a_ref, b_ref, oa_ref, ob_ref):
        oa_ref[...] = a_ref[...].astype(jnp.bfloat16)
        ob_ref[...] = b_ref[...].astype(jnp.bfloat16)

    blk = D // 4
    return pl.pallas_call(
        body,
        grid=(4,),
        in_specs=[pl.BlockSpec((blk, D), lambda i: (i, 0)),
                  pl.BlockSpec((blk, D), lambda i: (i, 0))],
        out_specs=[pl.BlockSpec((blk, D), lambda i: (i, 0)),
                   pl.BlockSpec((blk, D), lambda i: (i, 0))],
        out_shape=[jax.ShapeDtypeStruct((D, D), jnp.bfloat16)] * 2,
    )(w0, w1)


def _routed_matmul_split(flags, x_sc, x_tc, w0, w1):
    kt = x_sc.shape[0] // T
    ntiles = kt + x_tc.shape[0] // T

    def body(flags_ref, xsc_ref, xtc_ref, w0_ref, w1_ref, o_ref):
        t = pl.program_id(0)
        f = flags_ref[t]
        xb = jnp.where(t < kt, xsc_ref[...], xtc_ref[...]).astype(jnp.bfloat16)

        @pl.when(f == 0)
        def _():
            o_ref[...] = lax.dot_general(
                xb, w0_ref[...], (((1,), (1,)), ((), ())),
                preferred_element_type=jnp.float32)

        @pl.when(f != 0)
        def _():
            o_ref[...] = lax.dot_general(
                xb, w1_ref[...], (((1,), (1,)), ((), ())),
                preferred_element_type=jnp.float32)

    grid_spec = pltpu.PrefetchScalarGridSpec(
        num_scalar_prefetch=1,
        grid=(ntiles,),
        in_specs=[
            pl.BlockSpec((T, D), lambda t, flags: (jnp.minimum(t, kt - 1), 0)),
            pl.BlockSpec((T, D), lambda t, flags: (jnp.maximum(t - kt, 0), 0)),
            pl.BlockSpec((D, D), lambda t, flags: (0, 0)),
            pl.BlockSpec((D, D), lambda t, flags: (0, 0)),
        ],
        out_specs=pl.BlockSpec((T, D), lambda t, flags: (t, 0)),
    )
    return pl.pallas_call(
        body,
        grid_spec=grid_spec,
        out_shape=jax.ShapeDtypeStruct((ntiles * T, D), jnp.float32),
    )(flags, x_sc, x_tc, w0, w1)


def _routed_matmul(flags, x, w0, w1):
    n = x.shape[0]
    ntiles = n // T

    def body(flags_ref, x_ref, w0_ref, w1_ref, o_ref):
        f = flags_ref[pl.program_id(0)]
        xb = x_ref[...].astype(jnp.bfloat16)

        @pl.when(f == 0)
        def _():
            o_ref[...] = lax.dot_general(
                xb, w0_ref[...], (((1,), (1,)), ((), ())),
                preferred_element_type=jnp.float32)

        @pl.when(f != 0)
        def _():
            o_ref[...] = lax.dot_general(
                xb, w1_ref[...], (((1,), (1,)), ((), ())),
                preferred_element_type=jnp.float32)

    grid_spec = pltpu.PrefetchScalarGridSpec(
        num_scalar_prefetch=1,
        grid=(ntiles,),
        in_specs=[
            pl.BlockSpec((T, D), lambda t, flags: (t, 0)),
            pl.BlockSpec((D, D), lambda t, flags: (0, 0)),
            pl.BlockSpec((D, D), lambda t, flags: (0, 0)),
        ],
        out_specs=pl.BlockSpec((T, D), lambda t, flags: (t, 0)),
    )
    return pl.pallas_call(
        body,
        grid_spec=grid_spec,
        out_shape=jax.ShapeDtypeStruct((n, D), jnp.float32),
    )(flags, x, w0, w1)


S = 2


def kernel(input_ids, role_mask, table, W0, W1):
    b, l = input_ids.shape
    n = b * l
    seg = n // S
    npad = seg + T
    ntiles = npad // T

    ids = input_ids.reshape(S, seg).astype(jnp.int32)
    is0 = role_mask.reshape(S, seg) == 0
    i0 = is0.astype(jnp.int32)
    r0 = jnp.cumsum(i0, axis=1) - 1
    iota = jnp.arange(seg, dtype=jnp.int32)[None, :]
    c0 = r0[:, -1:] + 1
    start1 = (c0 // T) * T + T
    pos = jnp.where(is0, r0, start1 + iota - r0 - 1).astype(jnp.int32)
    flags = (jnp.arange(ntiles, dtype=jnp.int32)[None, :] >= (c0 // T + 1)).astype(jnp.int32)

    w0b, w1b = _bf16_cast_pair(W0, W1)

    out_ref = jax.new_ref(lax.empty((n, D), jnp.float32))
    for s in range(S):
        x_s = _make_row_permute_gather(seg, npad, D, 16)(
            table, ids[s].reshape(NW, -1, 16), pos[s].reshape(NW, -1, 16))
        y_s = _routed_matmul(flags[s], x_s, w0b, w1b)
        _make_row_gather_into(seg, D, 16, s * seg)(
            y_s, pos[s].reshape(NW, -1, 16), out_ref)

    return out_ref[...].reshape(b, l, D)
